# baseline (speedup 1.0000x reference)
import functools

import jax
import jax.numpy as jnp
from jax import lax
from jax.experimental import pallas as pl
from jax.experimental.pallas import tpu as pltpu
from jax.experimental.pallas import tpu_sc as plsc

_NC = 2
_NS = 16
_NW = _NC * _NS
_K = 128
_NBUF = 4
_RB = 1000


def _sc_mesh():
    return plsc.VectorSubcoreMesh(core_axis_name="c", subcore_axis_name="s",
                                  num_cores=_NC, num_subcores=_NS)


@functools.lru_cache(maxsize=None)
def _make_deg_kernel(n_pad, e_pad):
    npt = n_pad // _NS
    epw = e_pad // _NW
    nch = epw // _K
    assert npt * _NS == n_pad and npt % 8 == 0
    assert epw * _NW == e_pad and nch * _K == epw

    @functools.partial(
        pl.kernel,
        out_type=jax.ShapeDtypeStruct((_NC * n_pad, 8), jnp.float32),
        mesh=_sc_mesh(),
        scratch_types=[
            pltpu.VMEM((_K,), jnp.int32),
            pltpu.VMEM((_K, 8), jnp.float32),
            pltpu.VMEM_SHARED((n_pad, 8), jnp.float32),
        ],
    )
    def deg_kernel(dst_hbm, zeros_hbm, ones_hbm, out_hbm,
                   didx, ones_v, acc_sh):
        cid = lax.axis_index("c")
        sid = lax.axis_index("s")
        w = cid * _NS + sid
        rbase = sid * npt
        pltpu.sync_copy(zeros_hbm.at[pl.ds(rbase, npt)],
                        acc_sh.at[pl.ds(rbase, npt)])
        pltpu.sync_copy(ones_hbm, ones_v)
        plsc.subcore_barrier()

        @pl.loop(0, nch)
        def _(j):
            pltpu.sync_copy(dst_hbm.at[pl.ds(w * epw + j * _K, _K)], didx)
            pltpu.sync_copy(ones_v, acc_sh.at[didx], add=True)

        plsc.subcore_barrier()
        pltpu.sync_copy(acc_sh.at[pl.ds(rbase, npt)],
                        out_hbm.at[pl.ds(cid * n_pad + rbase, npt)])

    return deg_kernel


@functools.lru_cache(maxsize=None)
def _make_agg_kernel(n_pad, e_pad, hw):
    npt = n_pad // _NS
    epw = e_pad // _NW
    nch = epw // _K
    assert npt * _NS == n_pad and npt % 8 == 0
    assert epw * _NW == e_pad and nch * _K == epw and nch % _NBUF == 0

    @functools.partial(
        pl.kernel,
        out_type=jax.ShapeDtypeStruct((_NC * n_pad, hw), jnp.float32),
        mesh=_sc_mesh(),
        scratch_types=(
            [pltpu.VMEM((epw,), jnp.int32),
             pltpu.VMEM((_K,), jnp.int32),
             pltpu.VMEM((_K, hw), jnp.float32),
             pltpu.VMEM_SHARED((n_pad, hw), jnp.float32),
             pltpu.SemaphoreType.DMA,
             pltpu.SemaphoreType.DMA]
        ),
    )
    def agg_kernel(hp_hbm, src_hbm, dst_hbm, zeros_hbm, out_hbm,
                   sidx_all, didx, rbuf, acc_sh, gsem, dsem):
        cid = lax.axis_index("c")
        sid = lax.axis_index("s")
        w = cid * _NS + sid
        rbase = sid * npt
        pltpu.sync_copy(zeros_hbm.at[pl.ds(rbase, npt)],
                        acc_sh.at[pl.ds(rbase, npt)])
        pltpu.sync_copy(src_hbm.at[pl.ds(w * epw, epw)], sidx_all)
        plsc.subcore_barrier()

        @pl.loop(0, nch)
        def _(j):
            pltpu.async_copy(dst_hbm.at[pl.ds(w * epw + j * _K, _K)],
                             didx, dsem)
            pltpu.async_copy(hp_hbm.at[sidx_all.at[pl.ds(j * _K, _K)]],
                             rbuf, gsem)
            pltpu.make_async_copy(dst_hbm.at[pl.ds(0, _K)], didx, dsem).wait()
            pltpu.make_async_copy(hp_hbm.at[sidx_all.at[pl.ds(0, _K)]],
                                  rbuf, gsem).wait()
            pltpu.sync_copy(rbuf, acc_sh.at[didx], add=True)
        plsc.subcore_barrier()
        pltpu.sync_copy(acc_sh.at[pl.ds(rbase, npt)],
                        out_hbm.at[pl.ds(cid * n_pad + rbase, npt)])

    return agg_kernel


def _tc1_body(dp0_ref, dp1_ref, x_ref, w_ref, dinv_ref, hp_ref):
    indeg = dp0_ref[:, 0:1] + dp1_ref[:, 0:1]
    dinv = lax.rsqrt(indeg + 1.0)
    dinv_ref[...] = dinv
    hp = dinv * jnp.dot(x_ref[...], w_ref[...],
                        preferred_element_type=jnp.float32)
    hp_ref[...] = jnp.pad(hp, ((0, 0), (0, hp_ref.shape[1] - hp.shape[1])))


def _tc_mid_body(a0_ref, a1_ref, hp_ref, dinv_ref, b_ref, w_ref, out_ref):
    dinv = dinv_ref[...]
    h = w_ref.shape[0]
    z = dinv * (a0_ref[:, :h] + a1_ref[:, :h] + hp_ref[:, :h]) + b_ref[...]
    hrelu = jnp.maximum(z, 0.0)
    hp = dinv * jnp.dot(hrelu, w_ref[...],
                        preferred_element_type=jnp.float32)
    out_ref[...] = jnp.pad(hp, ((0, 0), (0, out_ref.shape[1] - hp.shape[1])))


def _tc_final_body(a0_ref, a1_ref, hp_ref, dinv_ref, b_ref, batch_ref,
                   wl_ref, bl_ref, out_ref, sums, counts):
    i = pl.program_id(0)
    nsteps = pl.num_programs(0)

    @pl.when(i == 0)
    def _():
        sums[...] = jnp.zeros_like(sums)
        counts[...] = jnp.zeros_like(counts)

    h = b_ref.shape[1]
    z = dinv_ref[...] * (a0_ref[:, :h] + a1_ref[:, :h] + hp_ref[:, :h]) + b_ref[...]
    seg = lax.broadcasted_iota(jnp.int32, (1, sums.shape[0]), 1)
    m = (batch_ref[...] == seg).astype(jnp.float32)
    dn = (((0,), (0,)), ((), ()))
    sums[...] += lax.dot_general(m, z, dn,
                                 preferred_element_type=jnp.float32)
    ones_col = jnp.ones((z.shape[0], 1), jnp.float32)
    counts[...] += lax.dot_general(m, ones_col, dn,
                                   preferred_element_type=jnp.float32)

    @pl.when(i == nsteps - 1)
    def _():
        pooled = sums[...] / jnp.maximum(counts[...], 1.0)
        out_ref[...] = jnp.dot(pooled, wl_ref[...],
                               preferred_element_type=jnp.float32) + bl_ref[...]


def kernel(x, edge_index, batch, W1, b1, W2, b2, W3, b3, Wl, bl):
    n, f_in = x.shape
    e = edge_index.shape[1]
    h = W1.shape[1]
    num_graphs = 64

    src = edge_index[0].astype(jnp.int32)
    dst = edge_index[1].astype(jnp.int32)
    batch2d = batch.astype(jnp.int32).reshape(n, 1)

    n_pad = ((n + _NS * 8 - 1) // (_NS * 8)) * (_NS * 8)
    hw = 128

    epw = -(-e // _NW)
    nch = -(-(-(-epw // _K)) // _NBUF) * _NBUF
    e_pad = _NW * nch * _K
    if e_pad > e and n_pad == n:
        n_pad += _NS * 8
    pad = e_pad - e
    src_p = jnp.concatenate([src, jnp.zeros((pad,), jnp.int32)])
    dst_p = jnp.concatenate([dst, jnp.full((pad,), n, jnp.int32)])

    zeros8 = jnp.zeros((n_pad, 8), jnp.float32)
    zeros_hw = jnp.zeros((n_pad, hw), jnp.float32)
    ones8 = jnp.ones((_K, 8), jnp.float32)

    degp = _make_deg_kernel(n_pad, e_pad)(dst_p, zeros8, ones8)
    dp0, dp1 = degp[:n], degp[n_pad:n_pad + n]

    grid = (n // _RB,)
    rb_spec8 = pl.BlockSpec((_RB, 8), lambda i: (i, 0))
    rb_spec1 = pl.BlockSpec((_RB, 1), lambda i: (i, 0))
    rb_spechw = pl.BlockSpec((_RB, hw), lambda i: (i, 0))
    full = lambda s: pl.BlockSpec(s, lambda i: tuple(0 for _ in s))

    dinv, hp1 = pl.pallas_call(
        _tc1_body,
        grid=grid,
        in_specs=[rb_spec8, rb_spec8,
                  pl.BlockSpec((_RB, f_in), lambda i: (i, 0)),
                  full((f_in, h))],
        out_specs=[rb_spec1, rb_spechw],
        out_shape=[jax.ShapeDtypeStruct((n, 1), jnp.float32),
                   jax.ShapeDtypeStruct((n, hw), jnp.float32)],
    )(dp0, dp1, x, W1)

    agg_fn = _make_agg_kernel(n_pad, e_pad, hw)

    def mid_layer(hp, bias, w_next):
        ap = agg_fn(hp, src_p, dst_p, zeros_hw)
        return pl.pallas_call(
            _tc_mid_body,
            grid=grid,
            in_specs=[rb_spechw, rb_spechw, rb_spechw, rb_spec1,
                      full((1, h)), full((h, h))],
            out_specs=rb_spechw,
            out_shape=jax.ShapeDtypeStruct((n, hw), jnp.float32),
        )(ap[:n], ap[n_pad:n_pad + n], hp, dinv, bias.reshape(1, h), w_next)

    hp2 = mid_layer(hp1, b1, W2)
    hp3 = mid_layer(hp2, b2, W3)

    ap3 = agg_fn(hp3, src_p, dst_p, zeros_hw)
    out = pl.pallas_call(
        _tc_final_body,
        grid=grid,
        in_specs=[rb_spechw, rb_spechw, rb_spechw, rb_spec1,
                  full((1, h)),
                  pl.BlockSpec((_RB, 1), lambda i: (i, 0)),
                  full((h, 1)), full((1, 1))],
        out_specs=pl.BlockSpec((num_graphs, 1), lambda i: (0, 0)),
        out_shape=jax.ShapeDtypeStruct((num_graphs, 1), jnp.float32),
        scratch_shapes=[pltpu.VMEM((num_graphs, h), jnp.float32),
                        pltpu.VMEM((num_graphs, 1), jnp.float32)],
    )(ap3[:n], ap3[n_pad:n_pad + n], hp3, dinv, b3.reshape(1, h), batch2d,
      Wl, bl.reshape(1, 1))
    return out


# speedup vs baseline: 2.4043x; 2.4043x over previous
import functools

import jax
import jax.numpy as jnp
from jax import lax
from jax.experimental import pallas as pl
from jax.experimental.pallas import tpu as pltpu
from jax.experimental.pallas import tpu_sc as plsc

_NC = 2
_NS = 16
_NW = _NC * _NS
_K = 128
_NBUF = 4
_RB = 1000


def _sc_mesh():
    return plsc.VectorSubcoreMesh(core_axis_name="c", subcore_axis_name="s",
                                  num_cores=_NC, num_subcores=_NS)


@functools.lru_cache(maxsize=None)
def _make_deg_kernel(n_pad, e_pad):
    npt = n_pad // _NS
    epw = e_pad // _NW
    nch = epw // _K
    assert npt * _NS == n_pad and npt % 8 == 0
    assert epw * _NW == e_pad and nch * _K == epw

    @functools.partial(
        pl.kernel,
        out_type=jax.ShapeDtypeStruct((_NC * n_pad, 8), jnp.float32),
        mesh=_sc_mesh(),
        scratch_types=[
            pltpu.VMEM((_K,), jnp.int32),
            pltpu.VMEM((_K, 8), jnp.float32),
            pltpu.VMEM_SHARED((n_pad, 8), jnp.float32),
        ],
    )
    def deg_kernel(dst_hbm, zeros_hbm, ones_hbm, out_hbm,
                   didx, ones_v, acc_sh):
        cid = lax.axis_index("c")
        sid = lax.axis_index("s")
        w = cid * _NS + sid
        rbase = sid * npt
        pltpu.sync_copy(zeros_hbm.at[pl.ds(rbase, npt)],
                        acc_sh.at[pl.ds(rbase, npt)])
        pltpu.sync_copy(ones_hbm, ones_v)
        plsc.subcore_barrier()

        @pl.loop(0, nch)
        def _(j):
            pltpu.sync_copy(dst_hbm.at[pl.ds(w * epw + j * _K, _K)], didx)
            pltpu.sync_copy(ones_v, acc_sh.at[didx], add=True)

        plsc.subcore_barrier()
        pltpu.sync_copy(acc_sh.at[pl.ds(rbase, npt)],
                        out_hbm.at[pl.ds(cid * n_pad + rbase, npt)])

    return deg_kernel


@functools.lru_cache(maxsize=None)
def _make_agg_kernel(n_pad, e_pad, hw):
    npt = n_pad // _NS
    epw = e_pad // _NW
    nch = epw // _K
    assert npt * _NS == n_pad and npt % 8 == 0
    assert epw * _NW == e_pad and nch * _K == epw

    @functools.partial(
        pl.kernel,
        out_type=jax.ShapeDtypeStruct((_NC * n_pad, hw), jnp.float32),
        mesh=_sc_mesh(),
        scratch_types=(
            [pltpu.VMEM((epw,), jnp.int32),
             pltpu.VMEM((_K,), jnp.int32),
             pltpu.VMEM((_K, hw), jnp.float32),
             pltpu.VMEM_SHARED((n_pad, hw), jnp.float32),
             pltpu.SemaphoreType.DMA,
             pltpu.SemaphoreType.DMA]
        ),
    )
    def agg_kernel(hp_hbm, src_hbm, dst_hbm, zeros_hbm, out_hbm,
                   sidx_all, didx, rbuf, acc_sh, gsem, dsem):
        cid = lax.axis_index("c")
        sid = lax.axis_index("s")
        w = cid * _NS + sid
        rbase = sid * npt
        pltpu.sync_copy(zeros_hbm.at[pl.ds(rbase, npt)],
                        acc_sh.at[pl.ds(rbase, npt)])
        pltpu.sync_copy(src_hbm.at[pl.ds(w * epw, epw)], sidx_all)
        plsc.subcore_barrier()

        @pl.loop(0, nch)
        def _(j):
            pltpu.async_copy(dst_hbm.at[pl.ds(w * epw + j * _K, _K)],
                             didx, dsem)
            pltpu.async_copy(hp_hbm.at[sidx_all.at[pl.ds(j * _K, _K)]],
                             rbuf, gsem)
            pltpu.make_async_copy(dst_hbm.at[pl.ds(0, _K)], didx, dsem).wait()
            pltpu.make_async_copy(hp_hbm.at[sidx_all.at[pl.ds(0, _K)]],
                                  rbuf, gsem).wait()
            pltpu.sync_copy(rbuf, acc_sh.at[didx], add=True)
        plsc.subcore_barrier()
        pltpu.sync_copy(acc_sh.at[pl.ds(rbase, npt)],
                        out_hbm.at[pl.ds(cid * n_pad + rbase, npt)])

    return agg_kernel


def _tc1_body(dp0_ref, dp1_ref, x_ref, w_ref, dinv_ref, hp_ref):
    indeg = dp0_ref[:, 0:1] + dp1_ref[:, 0:1]
    dinv = lax.rsqrt(indeg + 1.0)
    dinv_ref[...] = dinv
    hp = dinv * jnp.dot(x_ref[...], w_ref[...],
                        preferred_element_type=jnp.float32)
    hp_ref[...] = jnp.pad(hp, ((0, 0), (0, hp_ref.shape[1] - hp.shape[1])))


def _tc_mid_body(a0_ref, a1_ref, hp_ref, dinv_ref, b_ref, w_ref, out_ref):
    dinv = dinv_ref[...]
    h = w_ref.shape[0]
    z = dinv * (a0_ref[:, :h] + a1_ref[:, :h] + hp_ref[:, :h]) + b_ref[...]
    hrelu = jnp.maximum(z, 0.0)
    hp = dinv * jnp.dot(hrelu, w_ref[...],
                        preferred_element_type=jnp.float32)
    out_ref[...] = jnp.pad(hp, ((0, 0), (0, out_ref.shape[1] - hp.shape[1])))


def _tc_final_body(a0_ref, a1_ref, hp_ref, dinv_ref, b_ref, batch_ref,
                   wl_ref, bl_ref, out_ref, sums, counts):
    i = pl.program_id(0)
    nsteps = pl.num_programs(0)

    @pl.when(i == 0)
    def _():
        sums[...] = jnp.zeros_like(sums)
        counts[...] = jnp.zeros_like(counts)

    h = b_ref.shape[1]
    z = dinv_ref[...] * (a0_ref[:, :h] + a1_ref[:, :h] + hp_ref[:, :h]) + b_ref[...]
    seg = lax.broadcasted_iota(jnp.int32, (1, sums.shape[0]), 1)
    m = (batch_ref[...] == seg).astype(jnp.float32)
    dn = (((0,), (0,)), ((), ()))
    sums[...] += lax.dot_general(m, z, dn,
                                 preferred_element_type=jnp.float32)
    ones_col = jnp.ones((z.shape[0], 1), jnp.float32)
    counts[...] += lax.dot_general(m, ones_col, dn,
                                   preferred_element_type=jnp.float32)

    @pl.when(i == nsteps - 1)
    def _():
        pooled = sums[...] / jnp.maximum(counts[...], 1.0)
        out_ref[...] = jnp.dot(pooled, wl_ref[...],
                               preferred_element_type=jnp.float32) + bl_ref[...]


def kernel(x, edge_index, batch, W1, b1, W2, b2, W3, b3, Wl, bl):
    n, f_in = x.shape
    e = edge_index.shape[1]
    h = W1.shape[1]
    num_graphs = 64

    src = edge_index[0].astype(jnp.int32)
    dst = edge_index[1].astype(jnp.int32)
    batch2d = batch.astype(jnp.int32).reshape(n, 1)

    n_pad = ((n + _NS * 8 - 1) // (_NS * 8)) * (_NS * 8)
    hw = 128

    epw = -(-e // _NW)
    nch = -(-epw // _K)
    e_pad = _NW * nch * _K
    if e_pad > e and n_pad == n:
        n_pad += _NS * 8
    pad = e_pad - e
    pad_ar = jnp.arange(pad, dtype=jnp.int32)
    src_p = jnp.concatenate([src, pad_ar % jnp.int32(n)])
    dst_p = jnp.concatenate([dst, n + pad_ar % jnp.int32(n_pad - n)])

    zeros8 = jnp.zeros((n_pad, 8), jnp.float32)
    zeros_hw = jnp.zeros((n_pad, hw), jnp.float32)
    ones8 = jnp.ones((_K, 8), jnp.float32)

    degp = _make_deg_kernel(n_pad, e_pad)(dst_p, zeros8, ones8)
    dp0, dp1 = degp[:n], degp[n_pad:n_pad + n]

    grid = (n // _RB,)
    rb_spec8 = pl.BlockSpec((_RB, 8), lambda i: (i, 0))
    rb_spec1 = pl.BlockSpec((_RB, 1), lambda i: (i, 0))
    rb_spechw = pl.BlockSpec((_RB, hw), lambda i: (i, 0))
    full = lambda s: pl.BlockSpec(s, lambda i: tuple(0 for _ in s))

    dinv, hp1 = pl.pallas_call(
        _tc1_body,
        grid=grid,
        in_specs=[rb_spec8, rb_spec8,
                  pl.BlockSpec((_RB, f_in), lambda i: (i, 0)),
                  full((f_in, h))],
        out_specs=[rb_spec1, rb_spechw],
        out_shape=[jax.ShapeDtypeStruct((n, 1), jnp.float32),
                   jax.ShapeDtypeStruct((n, hw), jnp.float32)],
    )(dp0, dp1, x, W1)

    agg_fn = _make_agg_kernel(n_pad, e_pad, hw)

    def mid_layer(hp, bias, w_next):
        ap = agg_fn(hp, src_p, dst_p, zeros_hw)
        return pl.pallas_call(
            _tc_mid_body,
            grid=grid,
            in_specs=[rb_spechw, rb_spechw, rb_spechw, rb_spec1,
                      full((1, h)), full((h, h))],
            out_specs=rb_spechw,
            out_shape=jax.ShapeDtypeStruct((n, hw), jnp.float32),
        )(ap[:n], ap[n_pad:n_pad + n], hp, dinv, bias.reshape(1, h), w_next)

    hp2 = mid_layer(hp1, b1, W2)
    hp3 = mid_layer(hp2, b2, W3)

    ap3 = agg_fn(hp3, src_p, dst_p, zeros_hw)
    out = pl.pallas_call(
        _tc_final_body,
        grid=grid,
        in_specs=[rb_spechw, rb_spechw, rb_spechw, rb_spec1,
                  full((1, h)),
                  pl.BlockSpec((_RB, 1), lambda i: (i, 0)),
                  full((h, 1)), full((1, 1))],
        out_specs=pl.BlockSpec((num_graphs, 1), lambda i: (0, 0)),
        out_shape=jax.ShapeDtypeStruct((num_graphs, 1), jnp.float32),
        scratch_shapes=[pltpu.VMEM((num_graphs, h), jnp.float32),
                        pltpu.VMEM((num_graphs, 1), jnp.float32)],
    )(ap3[:n], ap3[n_pad:n_pad + n], hp3, dinv, b3.reshape(1, h), batch2d,
      Wl, bl.reshape(1, 1))
    return out
